# Initial kernel scaffold; baseline (speedup 1.0000x reference)
import jax
import jax.numpy as jnp
from jax.experimental import pallas as pl


def kernel(x, edge_index, W1, b1, W2, b2):
    raise NotImplementedError("write your pallas kernel here")


# speedup vs baseline: 6.6460x; 6.6460x over previous
import functools

import jax
import jax.numpy as jnp
from jax import lax
from jax.experimental import pallas as pl
from jax.experimental.pallas import tpu as pltpu
from jax.experimental.pallas import tpu_sc as plsc

N = 10000
D = 128
E = 320000

NC = 2
NS = 16

EDGES_PER_TILE = E // (NC * NS)
CHUNK = 128
FULL_CHUNKS = EDGES_PER_TILE // CHUNK
TAIL = EDGES_PER_TILE - FULL_CHUNKS * CHUNK
ROWS_PER_TILE = 632
LAST_ROW0 = N - ROWS_PER_TILE


def _sc_agg_build():
  mesh = plsc.VectorSubcoreMesh(core_axis_name="c", subcore_axis_name="s")

  @functools.partial(
      pl.kernel,
      mesh=mesh,
      out_type=jax.ShapeDtypeStruct((NC, N, D), jnp.float32),
      scratch_types=[
          pltpu.VMEM((CHUNK,), jnp.int32),
          pltpu.VMEM((CHUNK,), jnp.int32),
          pltpu.VMEM((CHUNK, D), jnp.float32),
          pltpu.VMEM((TAIL,), jnp.int32),
          pltpu.VMEM((TAIL,), jnp.int32),
          pltpu.VMEM((TAIL, D), jnp.float32),
          pltpu.VMEM_SHARED((N, D), jnp.float32),
          pltpu.SemaphoreType.DMA,
      ],
  )
  def sc_agg(x_hbm, src_hbm, dst_hbm, out_hbm, srci, dsti, rows, srct, dstt,
             rowst, acc, sem):
    cid = lax.axis_index("c")
    sid = lax.axis_index("s")

    def zero_row(i, _):
      for j in range(D // 16):
        rows[i, pl.ds(j * 16, 16)] = jnp.zeros((16,), jnp.float32)
      return 0
    lax.fori_loop(0, CHUNK, zero_row, 0)

    row0 = jnp.minimum(sid * ROWS_PER_TILE, LAST_ROW0)
    n_zc = ROWS_PER_TILE // CHUNK
    zr = ROWS_PER_TILE - n_zc * CHUNK
    for z in range(n_zc):
      pltpu.sync_copy(rows, acc.at[pl.ds(row0 + z * CHUNK, CHUNK)])
    if zr:
      pltpu.sync_copy(rows.at[pl.ds(0, zr)],
                      acc.at[pl.ds(row0 + n_zc * CHUNK, zr)])
    plsc.subcore_barrier()

    base_e = (cid * NS + sid) * EDGES_PER_TILE

    def edge_chunk(t, _):
      off = base_e + t * CHUNK
      pltpu.sync_copy(src_hbm.at[pl.ds(off, CHUNK)], srci)
      pltpu.sync_copy(dst_hbm.at[pl.ds(off, CHUNK)], dsti)
      pltpu.async_copy(x_hbm.at[srci], rows, sem).wait()
      pltpu.sync_copy(rows, acc.at[dsti], add=True)
      return 0
    lax.fori_loop(0, FULL_CHUNKS, edge_chunk, 0)

    if TAIL:
      off = base_e + FULL_CHUNKS * CHUNK
      pltpu.sync_copy(src_hbm.at[pl.ds(off, TAIL)], srct)
      pltpu.sync_copy(dst_hbm.at[pl.ds(off, TAIL)], dstt)
      pltpu.async_copy(x_hbm.at[srct], rowst, sem).wait()
      pltpu.sync_copy(rowst, acc.at[dstt], add=True)

    plsc.subcore_barrier()

    pltpu.sync_copy(acc.at[pl.ds(row0, ROWS_PER_TILE)],
                    out_hbm.at[cid, pl.ds(row0, ROWS_PER_TILE)])

  return sc_agg


_sc_agg = _sc_agg_build()

ROW_BLK = 1000


def _mlp_body(x_ref, a0_ref, a1_ref, w1_ref, b1_ref, w2_ref, b2_ref, o_ref):
  h = x_ref[...] + a0_ref[...] + a1_ref[...]
  h = jnp.dot(h, w1_ref[...], preferred_element_type=jnp.float32) + b1_ref[...]
  h = jnp.where(h > 0, h, jnp.exp(h) - 1.0)
  o_ref[...] = (
      jnp.dot(h, w2_ref[...], preferred_element_type=jnp.float32) + b2_ref[...]
  )


def _mlp(x, a0, a1, W1, b1, W2, b2):
  grid = (N // ROW_BLK,)
  row_spec = pl.BlockSpec((ROW_BLK, D), lambda i: (i, 0))
  full_spec = pl.BlockSpec((D, D), lambda i: (0, 0))
  bias_spec = pl.BlockSpec((1, D), lambda i: (0, 0))
  return pl.pallas_call(
      _mlp_body,
      grid=grid,
      in_specs=[row_spec, row_spec, row_spec, full_spec, bias_spec,
                full_spec, bias_spec],
      out_specs=row_spec,
      out_shape=jax.ShapeDtypeStruct((N, D), jnp.float32),
  )(x, a0, a1, W1, b1.reshape(1, D), W2, b2.reshape(1, D))


@jax.jit
def kernel(x, edge_index, W1, b1, W2, b2):
  ei = edge_index.astype(jnp.int32)
  agg2 = _sc_agg(x, ei[0], ei[1])
  return _mlp(x, agg2[0], agg2[1], W1, b1, W2, b2)
